# baseline (speedup 1.0000x reference)
import functools

import jax
import jax.numpy as jnp
from jax import lax
from jax.experimental import pallas as pl
from jax.experimental.pallas import tpu as pltpu
from jax.experimental.pallas import tpu_sc as plsc

NV = 10000
NCN = 10000
E = 160000
D = 128
NB = 8
NCAND = 2000

NPAD = 10240
NROW = NPAD // 128
NWRK = 32
CH = 128
EROWS = E // CH
EROWS_PAD = 1280
NCHUNK = EROWS_PAD // NWRK
K5_ROWS = 10240
K5_PW = K5_ROWS // NWRK

@functools.lru_cache(maxsize=1)
def _mesh():
    return plsc.VectorSubcoreMesh(core_axis_name="c", subcore_axis_name="s")


def _wid():
    return lax.axis_index("c") * 16 + lax.axis_index("s")


def _zero_vmem_1d(ref, n):
    z = jnp.zeros((16,), jnp.float32)

    def body(i, _):
        ref[pl.ds(i * 16, 16)] = z
        return 0

    lax.fori_loop(0, n // 16, body, 0)


def _zero_vmem_2d(ref, rows):
    z = jnp.zeros((16,), jnp.float32)

    def body(i, _):
        for c in range(8):
            ref[i, pl.ds(c * 16, 16)] = z
        return 0

    lax.fori_loop(0, rows, body, 0)


@functools.lru_cache(maxsize=1)
def _k_counts_fn():
    return functools.partial(
        pl.kernel,
        out_type=jax.ShapeDtypeStruct((2 * 2 * NPAD,), jnp.float32),
        mesh=_mesh(),
        scratch_types=[
            pltpu.VMEM_SHARED((2 * NPAD,), jnp.float32),
            pltpu.VMEM((NCHUNK, CH), jnp.int32),
            pltpu.VMEM((NCHUNK, CH), jnp.int32),
            pltpu.VMEM((CH,), jnp.float32),
            pltpu.VMEM((2 * NPAD // 16,), jnp.float32),
            pltpu.SemaphoreType.DMA,
        ],
    )(_k_counts_body)


def _k_counts_body(dst2d, srcsh2d, out, tbl, dbuf, sbuf, ones, zbuf, sem):
    core = lax.axis_index("c")
    sid = lax.axis_index("s")
    w = _wid()
    seg = 2 * NPAD // 16

    _zero_vmem_1d(zbuf, seg)
    for v in range(CH // 16):
        ones[pl.ds(v * 16, 16)] = jnp.full((16,), 1.0, jnp.float32)
    pltpu.sync_copy(zbuf, tbl.at[pl.ds(sid * seg, seg)])
    plsc.subcore_barrier()

    pltpu.sync_copy(dst2d.at[pl.ds(w * NCHUNK, NCHUNK)], dbuf)
    pltpu.sync_copy(srcsh2d.at[pl.ds(w * NCHUNK, NCHUNK)], sbuf)

    def body(k, _):
        pltpu.sync_copy(ones.at[pl.ds(0, CH)], tbl.at[dbuf.at[k]], add=True)
        pltpu.sync_copy(ones.at[pl.ds(0, CH)], tbl.at[sbuf.at[k]], add=True)
        return 0

    lax.fori_loop(0, NCHUNK, body, 0)
    plsc.subcore_barrier()
    pltpu.sync_copy(tbl.at[pl.ds(sid * seg, seg)],
                    out.at[pl.ds(core * 2 * NPAD + sid * seg, seg)])


def _k_top8_body(cnts_ref, b_ref, rdeg_ref, rcnt_ref):
    dsum = cnts_ref[0, 0] + cnts_ref[1, 0]
    csum = cnts_ref[0, 1] + cnts_ref[1, 1]
    r = lax.broadcasted_iota(jnp.int32, (NROW, 128), 0)
    c = lax.broadcasted_iota(jnp.int32, (NROW, 128), 1)
    flat = r * 128 + c
    valid = flat < NV
    d = jnp.where(valid, dsum, -1.0)
    for i in range(NB):
        m = jnp.max(d)
        idx = jnp.min(jnp.where(d == m, flat, jnp.int32(2**30)))
        b_ref[0, i] = idx
        d = jnp.where(flat == idx, -2.0, d)
    rdeg_ref[...] = 1.0 / jnp.maximum(dsum, 1.0)
    rcnt_ref[...] = 1.0 / jnp.maximum(csum, 1.0)


def _k_top8(cnts):
    return pl.pallas_call(
        _k_top8_body,
        out_shape=[
            jax.ShapeDtypeStruct((1, NB), jnp.int32),
            jax.ShapeDtypeStruct((NROW, 128), jnp.float32),
            jax.ShapeDtypeStruct((NROW, 128), jnp.float32),
        ],
        out_specs=[
            pl.BlockSpec(memory_space=pltpu.SMEM),
            pl.BlockSpec((NROW, 128), lambda: (0, 0)),
            pl.BlockSpec((NROW, 128), lambda: (0, 0)),
        ],
        in_specs=[pl.BlockSpec((2, 2, NROW, 128), lambda: (0, 0, 0, 0))],
    )(cnts)


def _k_mlp_body(x_ref, sh_ref, sc_ref, w1_ref, b1_ref, w2_ref, b2_ref, o_ref):
    h = (x_ref[...] + sh_ref[...]) * sc_ref[...]
    h = jnp.maximum(jnp.dot(h, w1_ref[...], preferred_element_type=jnp.float32) + b1_ref[...], 0.0)
    o_ref[...] = jnp.maximum(jnp.dot(h, w2_ref[...], preferred_element_type=jnp.float32) + b2_ref[...], 0.0)


def _k_mlp(x, sh, sc, w1t, b1, w2t, b2):
    k = x.shape[1]
    blk = 512
    return pl.pallas_call(
        _k_mlp_body,
        grid=(NPAD // blk,),
        out_shape=jax.ShapeDtypeStruct((NPAD, D), jnp.float32),
        in_specs=[
            pl.BlockSpec((blk, k), lambda j: (j, 0)),
            pl.BlockSpec((1, k), lambda j: (0, 0)),
            pl.BlockSpec((1, k), lambda j: (0, 0)),
            pl.BlockSpec((k, D), lambda j: (0, 0)),
            pl.BlockSpec((1, D), lambda j: (0, 0)),
            pl.BlockSpec((D, D), lambda j: (0, 0)),
            pl.BlockSpec((1, D), lambda j: (0, 0)),
        ],
        out_specs=pl.BlockSpec((blk, D), lambda j: (j, 0)),
    )(x, sh, sc, w1t, b1, w2t, b2)


@functools.lru_cache(maxsize=1)
def _k_segsum_fn():
    return functools.partial(
        pl.kernel,
        out_type=jax.ShapeDtypeStruct((2 * NPAD, D), jnp.float32),
        mesh=_mesh(),
        scratch_types=[
            pltpu.VMEM_SHARED((NPAD, D), jnp.float32),
            pltpu.VMEM((NCHUNK, CH), jnp.int32),
            pltpu.VMEM((NCHUNK, CH), jnp.int32),
            pltpu.VMEM((CH, D), jnp.float32),
            pltpu.VMEM((CH, D), jnp.float32),
            pltpu.SemaphoreType.DMA,
            pltpu.SemaphoreType.DMA,
            pltpu.SemaphoreType.DMA,
            pltpu.SemaphoreType.DMA,
        ],
    )(_k_segsum_body)


def _k_segsum_body(table, gidx, sidx, out, acc, gbuf, sbuf, rows_a, rows_b,
                   gs_a, gs_b, ss_a, ss_b):
    core = lax.axis_index("c")
    sid = lax.axis_index("s")
    w = _wid()
    bufs = (rows_a, rows_b)
    gsems = (gs_a, gs_b)
    ssems = (ss_a, ss_b)

    _zero_vmem_2d(rows_a, CH)
    for q in range(NPAD // 16 // CH):
        pltpu.sync_copy(rows_a, acc.at[pl.ds(sid * (NPAD // 16) + q * CH, CH)])
    plsc.subcore_barrier()

    pltpu.sync_copy(gidx.at[pl.ds(w * NCHUNK, NCHUNK)], gbuf)
    pltpu.sync_copy(sidx.at[pl.ds(w * NCHUNK, NCHUNK)], sbuf)

    def body(k2, _):
        d0 = pltpu.async_copy(table.at[gbuf.at[2 * k2]], bufs[0], gsems[0])
        d1 = pltpu.async_copy(table.at[gbuf.at[2 * k2 + 1]], bufs[1], gsems[1])
        d0.wait()
        pltpu.sync_copy(bufs[0], acc.at[sbuf.at[2 * k2]], add=True)
        d1.wait()
        pltpu.sync_copy(bufs[1], acc.at[sbuf.at[2 * k2 + 1]], add=True)
        return 0

    lax.fori_loop(0, NCHUNK // 2, body, 0)
    plsc.subcore_barrier()
    for q in range(NPAD // 128 // 16):
        off = sid * (NPAD // 16) + q * 128
        pltpu.sync_copy(acc.at[pl.ds(off, 128)],
                        out.at[pl.ds(core * NPAD + off, 128)])


@functools.lru_cache(maxsize=1)
def _k_mtable_fn():
    return functools.partial(
        pl.kernel,
        out_type=jax.ShapeDtypeStruct((2 * NPAD * 8,), jnp.float32),
        mesh=_mesh(),
        scratch_types=[
            pltpu.VMEM_SHARED((NPAD * 8,), jnp.float32),
            pltpu.VMEM((K5_PW, 16), jnp.int32),
            pltpu.VMEM((K5_PW, 16), jnp.int32),
            pltpu.VMEM((16,), jnp.int32),
            pltpu.VMEM((8, 16), jnp.int32),
            pltpu.VMEM((8, 16), jnp.float32),
            pltpu.VMEM((NPAD * 8 // 16,), jnp.float32),
        ],
    )(_k_mtable_body)


def _k_mtable_body(src16, dst16, bvec, out, msh, sbuf, dbuf, bbuf, istg, vstg, zbuf):
    core = lax.axis_index("c")
    sid = lax.axis_index("s")
    w = _wid()
    seg = NPAD * 8 // 16

    _zero_vmem_1d(zbuf, seg)
    pltpu.sync_copy(zbuf, msh.at[pl.ds(sid * seg, seg)])
    pltpu.sync_copy(bvec, bbuf)
    plsc.subcore_barrier()

    pltpu.sync_copy(src16.at[pl.ds(w * K5_PW, K5_PW)], sbuf)
    pltpu.sync_copy(dst16.at[pl.ds(w * K5_PW, K5_PW)], dbuf)
    bb = bbuf[pl.ds(0, 16)]
    bs = [bb[i] for i in range(NB)]

    def body(k, _):
        dstv = dbuf[k, :]
        srcv = sbuf[k, :]
        hit = dstv == bs[0]
        for i in range(1, NB):
            hit = hit | (dstv == bs[i])
        h32 = jnp.where(hit, 1, 0)
        s = h32[0]
        for l in range(1, 16):
            s = s | h32[l]

        @pl.when(s > 0)
        def _rare():
            base8 = srcv * 8
            for i in range(NB):
                istg[i, :] = base8 + i
                vstg[i, :] = jnp.where(dstv == bs[i], 1.0, 0.0)
            for i in range(NB):
                pltpu.sync_copy(vstg.at[i], msh.at[istg.at[i]], add=True)

        return 0

    lax.fori_loop(0, K5_PW, body, 0)
    plsc.subcore_barrier()
    pltpu.sync_copy(msh.at[pl.ds(sid * seg, seg)],
                    out.at[pl.ds(core * NPAD * 8 + sid * seg, seg)])


def _k_layer1_body(bsm_ref, cand_ref, scv0a_ref, scv0b_ref, svc0a_ref, svc0b_ref,
                   var0_ref, cons0_ref, rdeg_ref, rcnt_ref, ma_ref, mb_ref,
                   w_ref, wl1cv_ref, wr1cv_ref, b1cv_ref, wl1vc_ref,
                   wr1vc_ref, b1vc_ref,
                   var1_ref, cons1_ref, delta_ref, dv1_ref, flag_ref, slot_ref):
    j = pl.program_id(0)
    blk = var0_ref.shape[0]

    a_cv0 = (scv0a_ref[...] + scv0b_ref[...]) * rdeg_ref[...]
    pre_v1 = (jnp.dot(a_cv0, wl1cv_ref[...], preferred_element_type=jnp.float32)
              + b1cv_ref[...]
              + jnp.dot(var0_ref[...], wr1cv_ref[...], preferred_element_type=jnp.float32))
    var1 = jnp.maximum(pre_v1, 0.0)
    var1_ref[...] = var1

    a_vc0 = (svc0a_ref[...] + svc0b_ref[...]) * rcnt_ref[...]
    pre_c1 = (jnp.dot(a_vc0, wl1vc_ref[...], preferred_element_type=jnp.float32)
              + b1vc_ref[...]
              + jnp.dot(cons0_ref[...], wr1vc_ref[...], preferred_element_type=jnp.float32))
    cons1 = jnp.maximum(pre_c1, 0.0)
    cons1_ref[...] = cons1

    u_c = jnp.dot(w_ref[...], wl1vc_ref[...], preferred_element_type=jnp.float32)
    u_v = jnp.dot(w_ref[...], wr1cv_ref[...], preferred_element_type=jnp.float32)

    m = ma_ref[...] + mb_ref[...]
    alpha = m * rcnt_ref[...]
    bits = jnp.zeros((blk, 1), jnp.int32)
    for i in range(NB):
        delta_ref[i] = jnp.maximum(pre_c1 + alpha[:, i:i + 1] * u_c, 0.0) - cons1
        bits = bits + jnp.where(m[:, i:i + 1] > 0.0, jnp.int32(1 << i), 0)
    flag_ref[...] = bits

    rowid = j * blk + lax.broadcasted_iota(jnp.int32, (blk, 1), 0)
    pos = lax.broadcasted_iota(jnp.int32, (1, 2048), 1)
    eq = rowid == cand_ref[...]
    slot_ref[...] = jnp.min(jnp.where(eq, pos, jnp.int32(2047)), axis=1, keepdims=True)

    dblk = jnp.maximum(pre_v1 + u_v, 0.0) - var1

    @pl.when(j == 0)
    def _init():
        dv1_ref[...] = jnp.zeros((NB, D), jnp.float32)

    for i in range(NB):
        sel = rowid == bsm_ref[0, i]
        contrib = jnp.sum(jnp.where(sel, dblk, 0.0), axis=0, keepdims=True)
        dv1_ref[pl.ds(i, 1), :] = dv1_ref[pl.ds(i, 1), :] + contrib


def _k_layer1(bsm, cand2048, scv0a, scv0b, svc0a, svc0b, var0, cons0, rdeg, rcnt,
              ma, mb, w, wl1cv, wr1cv, b1cv, wl1vc, wr1vc, b1vc):
    blk = 512
    g = NPAD // blk
    full = lambda shape: pl.BlockSpec(shape, lambda j: tuple(0 for _ in shape))
    rowblk = pl.BlockSpec((blk, D), lambda j: (j, 0))
    return pl.pallas_call(
        _k_layer1_body,
        grid=(g,),
        out_shape=[
            jax.ShapeDtypeStruct((NPAD, D), jnp.float32),
            jax.ShapeDtypeStruct((NPAD, D), jnp.float32),
            jax.ShapeDtypeStruct((NB, NPAD, D), jnp.float32),
            jax.ShapeDtypeStruct((NB, D), jnp.float32),
            jax.ShapeDtypeStruct((NPAD, 1), jnp.int32),
            jax.ShapeDtypeStruct((NPAD, 1), jnp.int32),
        ],
        in_specs=[
            pl.BlockSpec(memory_space=pltpu.SMEM),
            full((1, 2048)),
            rowblk, rowblk, rowblk, rowblk, rowblk, rowblk,
            pl.BlockSpec((blk, 1), lambda j: (j, 0)),
            pl.BlockSpec((blk, 1), lambda j: (j, 0)),
            pl.BlockSpec((blk, NB), lambda j: (j, 0)),
            pl.BlockSpec((blk, NB), lambda j: (j, 0)),
            full((1, D)), full((D, D)), full((D, D)), full((1, D)),
            full((D, D)), full((D, D)), full((1, D)),
        ],
        out_specs=[
            rowblk, rowblk,
            pl.BlockSpec((NB, blk, D), lambda j: (0, j, 0)),
            pl.BlockSpec((NB, D), lambda j: (0, 0)),
            pl.BlockSpec((blk, 1), lambda j: (j, 0)),
            pl.BlockSpec((blk, 1), lambda j: (j, 0)),
        ],
    )(bsm, cand2048, scv0a, scv0b, svc0a, svc0b, var0, cons0, rdeg, rcnt,
      ma, mb, w, wl1cv, wr1cv, b1cv, wl1vc, wr1vc, b1vc)


def _k_pre2_body(sa_ref, sb_ref, var1_ref, rdeg_ref, wl2_ref, wr2_ref, b2_ref, o_ref):
    a = (sa_ref[...] + sb_ref[...]) * rdeg_ref[...]
    o_ref[...] = (jnp.dot(a, wl2_ref[...], preferred_element_type=jnp.float32)
                  + b2_ref[...]
                  + jnp.dot(var1_ref[...], wr2_ref[...], preferred_element_type=jnp.float32))


def _k_pre2(sa, sb, var1, rdeg, wl2t, wr2t, b2):
    blk = 512
    full = lambda shape: pl.BlockSpec(shape, lambda j: tuple(0 for _ in shape))
    rowblk = pl.BlockSpec((blk, D), lambda j: (j, 0))
    return pl.pallas_call(
        _k_pre2_body,
        grid=(NPAD // blk,),
        out_shape=jax.ShapeDtypeStruct((NPAD, D), jnp.float32),
        in_specs=[rowblk, rowblk, rowblk,
                  pl.BlockSpec((blk, 1), lambda j: (j, 0)),
                  full((D, D)), full((D, D)), full((1, D))],
        out_specs=rowblk,
    )(sa, sb, var1, rdeg, wl2t, wr2t, b2)


SLOTS = 2048
WL_CAP = 440


@functools.lru_cache(maxsize=1)
def _k_scan_fn():
    return functools.partial(
        pl.kernel,
        out_type=jax.ShapeDtypeStruct((2 * 4 * SLOTS, D), jnp.float32),
        mesh=_mesh(),
        compiler_params=pltpu.CompilerParams(needs_layout_passes=False),
        scratch_types=[
            pltpu.VMEM_SHARED((4 * SLOTS, D), jnp.float32),
            pltpu.VMEM((NPAD // 1024, 8, 128), jnp.int32),
            pltpu.VMEM((NPAD // 1024, 8, 128), jnp.int32),
            pltpu.VMEM((10, 8, 128), jnp.int32),
            pltpu.VMEM((10, 8, 128), jnp.int32),
            pltpu.VMEM((520,), jnp.int32),
            pltpu.VMEM((520,), jnp.int32),
            pltpu.VMEM((16, D), jnp.float32),
            pltpu.SemaphoreType.DMA,
        ],
    )(_k_scan_body)


def _k_scan_body(src16, dst16, flagh, sloth, delta, out,
                 acc, flagv, slotv, sbuf, dbuf, wlg, wls, rows, sem):
    core = lax.axis_index("c")
    sid = lax.axis_index("s")

    _zero_vmem_2d(rows, 16)
    for q in range(4 * SLOTS // 16 // 16):
        pltpu.sync_copy(rows, acc.at[pl.ds(sid * (4 * SLOTS // 16) + q * 16, 16)])
    plsc.subcore_barrier()

    pltpu.sync_copy(flagh, flagv)
    pltpu.sync_copy(sloth, slotv)
    pltpu.sync_copy(src16.at[pl.ds(sid * 10, 10)], sbuf)
    pltpu.sync_copy(dst16.at[pl.ds(sid * 10, 10)], dbuf)

    ibase = core * 4

    def batch(t, _):
        gv = wlg[pl.ds(t * 16, 16)]
        sv = wls[pl.ds(t * 16, 16)]
        pltpu.async_copy(delta.at[gv], rows, sem).wait()
        pltpu.sync_copy(rows, acc.at[sv], add=True)
        return 0

    def drain(c):
        nb = c // 16
        lax.fori_loop(0, nb, batch, 0)
        tail_g = wlg[pl.ds(nb * 16, 16)]
        tail_s = wls[pl.ds(nb * 16, 16)]
        wlg[pl.ds(0, 16)] = tail_g
        wls[pl.ds(0, 16)] = tail_s
        return c - nb * 16

    def scan_row(k, cnt):
        srcv = sbuf[k >> 6, (k >> 3) & 7, pl.ds((k & 7) * 16, 16)]
        fv = plsc.load_gather(flagv, [srcv >> 10, (srcv >> 7) & 7, srcv & 127])
        anyf = fv[0]
        for l in range(1, 16):
            anyf = anyf | fv[l]

        def rare(c):
            dstv = dbuf[k >> 6, (k >> 3) & 7, pl.ds((k & 7) * 16, 16)]
            sl = plsc.load_gather(slotv, [dstv >> 10, (dstv >> 7) & 7, dstv & 127])
            for il in range(4):
                ig = ibase + il
                qual = ((fv >> ig) & 1) != 0
                q32 = jnp.where(qual, 1, 0)
                pc = q32[0]
                for l in range(1, 16):
                    pc = pc + q32[l]
                plsc.store_compressed(wlg.at[pl.ds(c, 16)], srcv + ig * NPAD, mask=qual)
                plsc.store_compressed(wls.at[pl.ds(c, 16)], sl + il * SLOTS, mask=qual)
                c = c + pc
            return c

        real = (sid * 640 + k) * 16 < E
        cnt = lax.cond(jnp.logical_and(real, anyf != 0), rare, lambda c: c, cnt)
        cnt = lax.cond(cnt >= WL_CAP, drain, lambda c: c, cnt)
        return cnt

    cnt = lax.fori_loop(0, 640, scan_row, jnp.int32(0))
    wlg[pl.ds(cnt, 16)] = jnp.zeros((16,), jnp.int32)
    wls[pl.ds(cnt, 16)] = jnp.full((16,), 4 * SLOTS - 1, jnp.int32)
    nb = (cnt + 15) // 16
    lax.fori_loop(0, nb, batch, 0)

    plsc.subcore_barrier()
    seg = 4 * SLOTS // 16
    for q in range(seg // 128):
        off = sid * seg + q * 128
        pltpu.sync_copy(acc.at[pl.ds(off, 128)],
                        out.at[pl.ds(core * 4 * SLOTS + off, 128)])


@functools.lru_cache(maxsize=1)
def _k_cgather_fn():
    return functools.partial(
        pl.kernel,
        out_type=(
            jax.ShapeDtypeStruct((NB * SLOTS, D), jnp.float32),
            jax.ShapeDtypeStruct((SLOTS, D), jnp.float32),
            jax.ShapeDtypeStruct((SLOTS,), jnp.float32),
        ),
        mesh=_mesh(),
        compiler_params=pltpu.CompilerParams(needs_layout_passes=False),
        scratch_types=[
            pltpu.VMEM((NPAD // 1024, 8, 128), jnp.int32),
            pltpu.VMEM((NPAD // 1024, 8, 128), jnp.float32),
            pltpu.VMEM((SLOTS // 16, 16), jnp.int32),
            pltpu.VMEM((16, D), jnp.float32),
            pltpu.VMEM((64,), jnp.float32),
            pltpu.SemaphoreType.DMA,
        ],
    )(_k_cgather_body)


def _k_cgather_body(dacc, pre2, sloth, rdegh, cand2d, og, op, or_,
                    slotv, rdegv, candv, rows, rstage, sem):
    w = _wid()
    pltpu.sync_copy(sloth, slotv)
    pltpu.sync_copy(rdegh, rdegv)
    pltpu.sync_copy(cand2d, candv)

    def chunk(ch, _):
        k = w * 4 + ch
        cv = candv[k, :]
        pltpu.async_copy(pre2.at[candv.at[k]], rows, sem).wait()
        pltpu.sync_copy(rows, op.at[pl.ds(k * 16, 16)])
        rv = plsc.load_gather(rdegv, [cv >> 10, (cv >> 7) & 7, cv & 127])
        rstage[pl.ds(ch * 16, 16)] = rv
        sl = plsc.load_gather(slotv, [cv >> 10, (cv >> 7) & 7, cv & 127])
        for i in range(NB):
            pltpu.async_copy(dacc.at[sl + i * SLOTS], rows, sem).wait()
            pltpu.sync_copy(rows, og.at[pl.ds(i * SLOTS + k * 16, 16)])
        return 0

    lax.fori_loop(0, 4, chunk, 0)
    pltpu.sync_copy(rstage, or_.at[pl.ds(w * 64, 64)])


def _k_final_body(bsm_ref, g_ref, p_ref, r_ref, c_ref, dv1_ref, wl2_ref,
                  wr2_ref, o_ref):
    dwr = jnp.dot(dv1_ref[...], wr2_ref[...], preferred_element_type=jnp.float32)
    base = p_ref[...]
    rd = r_ref[...]
    cv = c_ref[...]
    acc = jnp.zeros_like(base)
    for i in range(NB):
        di = jnp.dot(g_ref[i] * rd, wl2_ref[...], preferred_element_type=jnp.float32)
        pre = base + di + jnp.where(cv == bsm_ref[0, i], dwr[i:i + 1, :], 0.0)
        acc = acc + jnp.maximum(pre, 0.0)
    o_ref[...] = acc * (1.0 / NB)


def _k_final(bsm, g, p, r, c, dv1, wl2t, wr2t):
    blk = 256
    full = lambda shape: pl.BlockSpec(shape, lambda j: tuple(0 for _ in shape))
    return pl.pallas_call(
        _k_final_body,
        grid=(SLOTS // blk,),
        out_shape=jax.ShapeDtypeStruct((SLOTS, D), jnp.float32),
        in_specs=[
            pl.BlockSpec(memory_space=pltpu.SMEM),
            pl.BlockSpec((NB, blk, D), lambda j: (0, j, 0)),
            pl.BlockSpec((blk, D), lambda j: (j, 0)),
            pl.BlockSpec((blk, 1), lambda j: (j, 0)),
            pl.BlockSpec((blk, 1), lambda j: (j, 0)),
            full((NB, D)), full((D, D)), full((D, D)),
        ],
        out_specs=pl.BlockSpec((blk, D), lambda j: (j, 0)),
    )(bsm, g, p, r, c, dv1, wl2t, wr2t)


def kernel(variable_embeddings, candidate_indices, constraint_x, variable_x,
           edge_index, edge_attr, params):
    p = params
    src = edge_index[0].astype(jnp.int32)
    dst = edge_index[1].astype(jnp.int32)
    cand = candidate_indices.astype(jnp.int32)

    padrows = (EROWS_PAD - EROWS) * CH
    deadpad = jnp.full((padrows,), NPAD - 1, jnp.int32)
    src2d = jnp.concatenate([src, deadpad]).reshape(EROWS_PAD, CH)
    dst2d = jnp.concatenate([dst, deadpad]).reshape(EROWS_PAD, CH)
    srcsh2d = src2d + NPAD
    pad16 = K5_ROWS * 16 - E
    src16 = jnp.concatenate([src, jnp.zeros((pad16,), jnp.int32)]).reshape(K5_ROWS, 16)
    dst16 = jnp.concatenate([dst, jnp.full((pad16,), -1, jnp.int32)]).reshape(K5_ROWS, 16)

    def padx(x, k):
        n, f = x.shape
        return jnp.pad(x, ((0, NPAD - n), (0, k - f)))

    cx = padx(constraint_x, 8)
    vx = padx(variable_x, 24)
    csh = jnp.pad(p['cons_shift'], (0, 3)).reshape(1, 8)
    csc = jnp.pad(p['cons_scale'], (0, 3)).reshape(1, 8)
    vsh = jnp.pad(p['var_shift'], (0, 5)).reshape(1, 24)
    vsc = jnp.pad(p['var_scale'], (0, 5)).reshape(1, 24)
    cw1t = jnp.pad(p['cons_W1'].T, ((0, 3), (0, 0)))
    vw1t = jnp.pad(p['var_W1'].T, ((0, 5), (0, 0)))
    cb1 = p['cons_b1'].reshape(1, D)
    cb2 = p['cons_b2'].reshape(1, D)
    vb1 = p['var_b1'].reshape(1, D)
    vb2 = p['var_b2'].reshape(1, D)
    cw2t = p['cons_W2'].T
    vw2t = p['var_W2'].T
    L1, L2 = p['convs'][0], p['convs'][1]
    w = p['break_W'][:, 0].reshape(1, D)

    cnts = _k_counts_fn()(dst2d, srcsh2d)
    bsm, rdeg80, rcnt80 = _k_top8(cnts.reshape(2, 2, NROW, 128))
    rdeg = rdeg80.reshape(NPAD, 1)
    rcnt = rcnt80.reshape(NPAD, 1)
    b8 = bsm[0]
    bvec = jnp.concatenate([b8, jnp.full((8,), 2**30, jnp.int32)])

    cons0 = _k_mlp(cx, csh, csc, cw1t, cb1, cw2t, cb2)
    var0 = _k_mlp(vx, vsh, vsc, vw1t, vb1, vw2t, vb2)

    mparts = _k_mtable_fn()(src16, dst16, bvec)
    ma = mparts[:NPAD * 8].reshape(NPAD, 8)
    mb = mparts[NPAD * 8:].reshape(NPAD, 8)

    src2d_d, _ = lax.optimization_barrier((src2d, mparts))
    scv0 = _k_segsum_fn()(cons0, src2d_d, dst2d)
    dst2d_d, _ = lax.optimization_barrier((dst2d, scv0))
    svc0 = _k_segsum_fn()(var0, dst2d_d, src2d)

    candp = jnp.concatenate([cand, jnp.zeros((48,), jnp.int32)])
    var1, cons1, delta, dv1, flagc, slotmap = _k_layer1(
        bsm, candp.reshape(1, 2048),
        scv0[:NPAD], scv0[NPAD:], svc0[:NPAD], svc0[NPAD:], var0, cons0, rdeg, rcnt,
        ma, mb, w, L1['cv_Wl'].T, L1['cv_Wr'].T, L1['cv_b'].reshape(1, D),
        L1['vc_Wl'].T, L1['vc_Wr'].T, L1['vc_b'].reshape(1, D))

    scv1 = _k_segsum_fn()(cons1, src2d, dst2d)
    pre2 = _k_pre2(scv1[:NPAD], scv1[NPAD:], var1, rdeg,
                   L2['cv_Wl'].T, L2['cv_Wr'].T, L2['cv_b'].reshape(1, D))

    sloth, _ = lax.optimization_barrier((slotmap.reshape(NPAD // 1024, 8, 128), scv1))
    dacc = _k_scan_fn()(src16.reshape(160, 8, 128), dst16.reshape(160, 8, 128),
                        flagc.reshape(NPAD // 1024, 8, 128), sloth,
                        delta.reshape(NPAD * NB, D))

    g, gp, gr = _k_cgather_fn()(dacc, pre2, slotmap.reshape(NPAD // 1024, 8, 128),
                                rdeg80.reshape(NPAD // 1024, 8, 128), candp.reshape(SLOTS // 16, 16))
    res = _k_final(bsm, g.reshape(NB, SLOTS, D), gp, gr.reshape(SLOTS, 1),
                   candp.reshape(SLOTS, 1), dv1, L2['cv_Wl'].T, L2['cv_Wr'].T)
    return res[:NCAND]


# speedup vs baseline: 2.1369x; 2.1369x over previous
import functools

import jax
import jax.numpy as jnp
from jax import lax
from jax.experimental import pallas as pl
from jax.experimental.pallas import tpu as pltpu
from jax.experimental.pallas import tpu_sc as plsc

NV = 10000
NCN = 10000
E = 160000
D = 128
NB = 8
NCAND = 2000

NPAD = 10240
NROW = NPAD // 128
NWRK = 32
CH = 128
EROWS = E // CH
EROWS_PAD = 1280
NCHUNK = EROWS_PAD // NWRK
K5_ROWS = 10240
K5_PW = K5_ROWS // NWRK

@functools.lru_cache(maxsize=1)
def _mesh():
    return plsc.VectorSubcoreMesh(core_axis_name="c", subcore_axis_name="s")


def _wid():
    return lax.axis_index("c") * 16 + lax.axis_index("s")


def _zero_vmem_1d(ref, n):
    z = jnp.zeros((16,), jnp.float32)

    def body(i, _):
        ref[pl.ds(i * 16, 16)] = z
        return 0

    lax.fori_loop(0, n // 16, body, 0)


def _zero_vmem_2d(ref, rows):
    z = jnp.zeros((16,), jnp.float32)

    def body(i, _):
        for c in range(8):
            ref[i, pl.ds(c * 16, 16)] = z
        return 0

    lax.fori_loop(0, rows, body, 0)


@functools.lru_cache(maxsize=1)
def _k_counts_fn():
    return functools.partial(
        pl.kernel,
        out_type=jax.ShapeDtypeStruct((2 * 2 * NPAD,), jnp.float32),
        mesh=_mesh(),
        scratch_types=[
            pltpu.VMEM_SHARED((2 * NPAD,), jnp.float32),
            pltpu.VMEM((NCHUNK, CH), jnp.int32),
            pltpu.VMEM((NCHUNK, CH), jnp.int32),
            pltpu.VMEM((CH,), jnp.float32),
            pltpu.VMEM((2 * NPAD // 16,), jnp.float32),
            pltpu.SemaphoreType.DMA,
        ],
    )(_k_counts_body)


def _k_counts_body(dst2d, srcsh2d, out, tbl, dbuf, sbuf, ones, zbuf, sem):
    core = lax.axis_index("c")
    sid = lax.axis_index("s")
    w = _wid()
    seg = 2 * NPAD // 16

    _zero_vmem_1d(zbuf, seg)
    for v in range(CH // 16):
        ones[pl.ds(v * 16, 16)] = jnp.full((16,), 1.0, jnp.float32)
    pltpu.sync_copy(zbuf, tbl.at[pl.ds(sid * seg, seg)])
    plsc.subcore_barrier()

    pltpu.sync_copy(dst2d.at[pl.ds(w * NCHUNK, NCHUNK)], dbuf)
    pltpu.sync_copy(srcsh2d.at[pl.ds(w * NCHUNK, NCHUNK)], sbuf)

    def body(k, _):
        pltpu.sync_copy(ones.at[pl.ds(0, CH)], tbl.at[dbuf.at[k]], add=True)
        pltpu.sync_copy(ones.at[pl.ds(0, CH)], tbl.at[sbuf.at[k]], add=True)
        return 0

    lax.fori_loop(0, NCHUNK, body, 0)
    plsc.subcore_barrier()
    pltpu.sync_copy(tbl.at[pl.ds(sid * seg, seg)],
                    out.at[pl.ds(core * 2 * NPAD + sid * seg, seg)])


def _k_top8_body(cnts_ref, b_ref, rdeg_ref, rcnt_ref):
    dsum = cnts_ref[0, 0] + cnts_ref[1, 0]
    csum = cnts_ref[0, 1] + cnts_ref[1, 1]
    r = lax.broadcasted_iota(jnp.int32, (NROW, 128), 0)
    c = lax.broadcasted_iota(jnp.int32, (NROW, 128), 1)
    flat = r * 128 + c
    valid = flat < NV
    d = jnp.where(valid, dsum, -1.0)
    for i in range(NB):
        m = jnp.max(d)
        idx = jnp.min(jnp.where(d == m, flat, jnp.int32(2**30)))
        b_ref[0, i] = idx
        d = jnp.where(flat == idx, -2.0, d)
    rdeg_ref[...] = 1.0 / jnp.maximum(dsum, 1.0)
    rcnt_ref[...] = 1.0 / jnp.maximum(csum, 1.0)


def _k_top8(cnts):
    return pl.pallas_call(
        _k_top8_body,
        out_shape=[
            jax.ShapeDtypeStruct((1, NB), jnp.int32),
            jax.ShapeDtypeStruct((NROW, 128), jnp.float32),
            jax.ShapeDtypeStruct((NROW, 128), jnp.float32),
        ],
        out_specs=[
            pl.BlockSpec(memory_space=pltpu.SMEM),
            pl.BlockSpec((NROW, 128), lambda: (0, 0)),
            pl.BlockSpec((NROW, 128), lambda: (0, 0)),
        ],
        in_specs=[pl.BlockSpec((2, 2, NROW, 128), lambda: (0, 0, 0, 0))],
    )(cnts)


def _k_mlp_body(x_ref, sh_ref, sc_ref, w1_ref, b1_ref, w2_ref, b2_ref, o_ref):
    h = (x_ref[...] + sh_ref[...]) * sc_ref[...]
    h = jnp.maximum(jnp.dot(h, w1_ref[...], preferred_element_type=jnp.float32) + b1_ref[...], 0.0)
    o_ref[...] = jnp.maximum(jnp.dot(h, w2_ref[...], preferred_element_type=jnp.float32) + b2_ref[...], 0.0)


def _k_mlp(x, sh, sc, w1t, b1, w2t, b2):
    k = x.shape[1]
    blk = 512
    return pl.pallas_call(
        _k_mlp_body,
        grid=(NPAD // blk,),
        out_shape=jax.ShapeDtypeStruct((NPAD, D), jnp.float32),
        in_specs=[
            pl.BlockSpec((blk, k), lambda j: (j, 0)),
            pl.BlockSpec((1, k), lambda j: (0, 0)),
            pl.BlockSpec((1, k), lambda j: (0, 0)),
            pl.BlockSpec((k, D), lambda j: (0, 0)),
            pl.BlockSpec((1, D), lambda j: (0, 0)),
            pl.BlockSpec((D, D), lambda j: (0, 0)),
            pl.BlockSpec((1, D), lambda j: (0, 0)),
        ],
        out_specs=pl.BlockSpec((blk, D), lambda j: (j, 0)),
    )(x, sh, sc, w1t, b1, w2t, b2)


@functools.lru_cache(maxsize=1)
def _k_segsum_fn():
    return functools.partial(
        pl.kernel,
        out_type=jax.ShapeDtypeStruct((2 * NPAD, D), jnp.float32),
        mesh=_mesh(),
        scratch_types=[
            pltpu.VMEM_SHARED((NPAD, D), jnp.float32),
            pltpu.VMEM((NCHUNK, CH), jnp.int32),
            pltpu.VMEM((NCHUNK, CH), jnp.int32),
            pltpu.VMEM((CH, D), jnp.float32),
            pltpu.VMEM((CH, D), jnp.float32),
            pltpu.SemaphoreType.DMA,
            pltpu.SemaphoreType.DMA,
            pltpu.SemaphoreType.DMA,
            pltpu.SemaphoreType.DMA,
        ],
    )(_k_segsum_body)


def _k_segsum_body(table, gidx, sidx, out, acc, gbuf, sbuf, rows_a, rows_b,
                   gs_a, gs_b, ss_a, ss_b):
    core = lax.axis_index("c")
    sid = lax.axis_index("s")
    w = _wid()
    bufs = (rows_a, rows_b)
    gsems = (gs_a, gs_b)
    ssems = (ss_a, ss_b)

    _zero_vmem_2d(rows_a, CH)
    for q in range(NPAD // 16 // CH):
        pltpu.sync_copy(rows_a, acc.at[pl.ds(sid * (NPAD // 16) + q * CH, CH)])
    plsc.subcore_barrier()

    pltpu.sync_copy(gidx.at[pl.ds(w * NCHUNK, NCHUNK)], gbuf)
    pltpu.sync_copy(sidx.at[pl.ds(w * NCHUNK, NCHUNK)], sbuf)

    def body(k2, _):
        d0 = pltpu.async_copy(table.at[gbuf.at[2 * k2]], bufs[0], gsems[0])
        d1 = pltpu.async_copy(table.at[gbuf.at[2 * k2 + 1]], bufs[1], gsems[1])
        d0.wait()
        pltpu.sync_copy(bufs[0], acc.at[sbuf.at[2 * k2]], add=True)
        d1.wait()
        pltpu.sync_copy(bufs[1], acc.at[sbuf.at[2 * k2 + 1]], add=True)
        return 0

    lax.fori_loop(0, NCHUNK // 2, body, 0)
    plsc.subcore_barrier()
    for q in range(NPAD // 128 // 16):
        off = sid * (NPAD // 16) + q * 128
        pltpu.sync_copy(acc.at[pl.ds(off, 128)],
                        out.at[pl.ds(core * NPAD + off, 128)])


@functools.lru_cache(maxsize=1)
def _k_mtable_fn():
    return functools.partial(
        pl.kernel,
        out_type=jax.ShapeDtypeStruct((2 * NPAD * 8,), jnp.float32),
        mesh=_mesh(),
        scratch_types=[
            pltpu.VMEM_SHARED((NPAD * 8,), jnp.float32),
            pltpu.VMEM((K5_PW, 16), jnp.int32),
            pltpu.VMEM((K5_PW, 16), jnp.int32),
            pltpu.VMEM((16,), jnp.int32),
            pltpu.VMEM((8, 16), jnp.int32),
            pltpu.VMEM((8, 16), jnp.float32),
            pltpu.VMEM((NPAD * 8 // 16,), jnp.float32),
        ],
    )(_k_mtable_body)


def _k_mtable_body(src16, dst16, bvec, out, msh, sbuf, dbuf, bbuf, istg, vstg, zbuf):
    core = lax.axis_index("c")
    sid = lax.axis_index("s")
    w = _wid()
    seg = NPAD * 8 // 16

    _zero_vmem_1d(zbuf, seg)
    pltpu.sync_copy(zbuf, msh.at[pl.ds(sid * seg, seg)])
    pltpu.sync_copy(bvec, bbuf)
    plsc.subcore_barrier()

    pltpu.sync_copy(src16.at[pl.ds(w * K5_PW, K5_PW)], sbuf)
    pltpu.sync_copy(dst16.at[pl.ds(w * K5_PW, K5_PW)], dbuf)
    bb = bbuf[pl.ds(0, 16)]
    bs = [bb[i] for i in range(NB)]

    def body(k, _):
        dstv = dbuf[k, :]
        srcv = sbuf[k, :]
        hit = dstv == bs[0]
        for i in range(1, NB):
            hit = hit | (dstv == bs[i])
        h32 = jnp.where(hit, 1, 0)
        s = h32[0]
        for l in range(1, 16):
            s = s | h32[l]

        @pl.when(s > 0)
        def _rare():
            base8 = srcv * 8
            for i in range(NB):
                istg[i, :] = base8 + i
                vstg[i, :] = jnp.where(dstv == bs[i], 1.0, 0.0)
            for i in range(NB):
                pltpu.sync_copy(vstg.at[i], msh.at[istg.at[i]], add=True)

        return 0

    lax.fori_loop(0, K5_PW, body, 0)
    plsc.subcore_barrier()
    pltpu.sync_copy(msh.at[pl.ds(sid * seg, seg)],
                    out.at[pl.ds(core * NPAD * 8 + sid * seg, seg)])


def _k_layer1_body(bsm_ref, cand_ref, scv0a_ref, scv0b_ref, svc0a_ref, svc0b_ref,
                   var0_ref, cons0_ref, rdeg_ref, rcnt_ref, ma_ref, mb_ref,
                   w_ref, wl1cv_ref, wr1cv_ref, b1cv_ref, wl1vc_ref,
                   wr1vc_ref, b1vc_ref,
                   var1_ref, cons1_ref, delta_ref, dv1_ref, flag_ref, slot_ref):
    j = pl.program_id(0)
    blk = var0_ref.shape[0]

    a_cv0 = (scv0a_ref[...] + scv0b_ref[...]) * rdeg_ref[...]
    pre_v1 = (jnp.dot(a_cv0, wl1cv_ref[...], preferred_element_type=jnp.float32)
              + b1cv_ref[...]
              + jnp.dot(var0_ref[...], wr1cv_ref[...], preferred_element_type=jnp.float32))
    var1 = jnp.maximum(pre_v1, 0.0)
    var1_ref[...] = var1

    a_vc0 = (svc0a_ref[...] + svc0b_ref[...]) * rcnt_ref[...]
    pre_c1 = (jnp.dot(a_vc0, wl1vc_ref[...], preferred_element_type=jnp.float32)
              + b1vc_ref[...]
              + jnp.dot(cons0_ref[...], wr1vc_ref[...], preferred_element_type=jnp.float32))
    cons1 = jnp.maximum(pre_c1, 0.0)
    cons1_ref[...] = cons1

    u_c = jnp.dot(w_ref[...], wl1vc_ref[...], preferred_element_type=jnp.float32)
    u_v = jnp.dot(w_ref[...], wr1cv_ref[...], preferred_element_type=jnp.float32)

    m = ma_ref[...] + mb_ref[...]
    alpha = m * rcnt_ref[...]
    bits = jnp.zeros((blk, 1), jnp.int32)
    for i in range(NB):
        delta_ref[i] = jnp.maximum(pre_c1 + alpha[:, i:i + 1] * u_c, 0.0) - cons1
        bits = bits + jnp.where(m[:, i:i + 1] > 0.0, jnp.int32(1 << i), 0)
    flag_ref[...] = bits

    rowid = j * blk + lax.broadcasted_iota(jnp.int32, (blk, 1), 0)
    pos = lax.broadcasted_iota(jnp.int32, (1, 2048), 1)
    eq = rowid == cand_ref[...]
    slot_ref[...] = jnp.min(jnp.where(eq, pos, jnp.int32(2047)), axis=1, keepdims=True)

    dblk = jnp.maximum(pre_v1 + u_v, 0.0) - var1

    @pl.when(j == 0)
    def _init():
        dv1_ref[...] = jnp.zeros((NB, D), jnp.float32)

    for i in range(NB):
        sel = rowid == bsm_ref[0, i]
        contrib = jnp.sum(jnp.where(sel, dblk, 0.0), axis=0, keepdims=True)
        dv1_ref[pl.ds(i, 1), :] = dv1_ref[pl.ds(i, 1), :] + contrib


def _k_layer1(bsm, cand2048, scv0a, scv0b, svc0a, svc0b, var0, cons0, rdeg, rcnt,
              ma, mb, w, wl1cv, wr1cv, b1cv, wl1vc, wr1vc, b1vc):
    blk = 512
    g = NPAD // blk
    full = lambda shape: pl.BlockSpec(shape, lambda j: tuple(0 for _ in shape))
    rowblk = pl.BlockSpec((blk, D), lambda j: (j, 0))
    return pl.pallas_call(
        _k_layer1_body,
        grid=(g,),
        out_shape=[
            jax.ShapeDtypeStruct((NPAD, D), jnp.float32),
            jax.ShapeDtypeStruct((NPAD, D), jnp.float32),
            jax.ShapeDtypeStruct((NB, NPAD, D), jnp.float32),
            jax.ShapeDtypeStruct((NB, D), jnp.float32),
            jax.ShapeDtypeStruct((NPAD, 1), jnp.int32),
            jax.ShapeDtypeStruct((NPAD, 1), jnp.int32),
        ],
        in_specs=[
            pl.BlockSpec(memory_space=pltpu.SMEM),
            full((1, 2048)),
            rowblk, rowblk, rowblk, rowblk, rowblk, rowblk,
            pl.BlockSpec((blk, 1), lambda j: (j, 0)),
            pl.BlockSpec((blk, 1), lambda j: (j, 0)),
            pl.BlockSpec((blk, NB), lambda j: (j, 0)),
            pl.BlockSpec((blk, NB), lambda j: (j, 0)),
            full((1, D)), full((D, D)), full((D, D)), full((1, D)),
            full((D, D)), full((D, D)), full((1, D)),
        ],
        out_specs=[
            rowblk, rowblk,
            pl.BlockSpec((NB, blk, D), lambda j: (0, j, 0)),
            pl.BlockSpec((NB, D), lambda j: (0, 0)),
            pl.BlockSpec((blk, 1), lambda j: (j, 0)),
            pl.BlockSpec((blk, 1), lambda j: (j, 0)),
        ],
    )(bsm, cand2048, scv0a, scv0b, svc0a, svc0b, var0, cons0, rdeg, rcnt,
      ma, mb, w, wl1cv, wr1cv, b1cv, wl1vc, wr1vc, b1vc)


def _k_pre2_body(sa_ref, sb_ref, var1_ref, rdeg_ref, wl2_ref, wr2_ref, b2_ref, o_ref):
    a = (sa_ref[...] + sb_ref[...]) * rdeg_ref[...]
    o_ref[...] = (jnp.dot(a, wl2_ref[...], preferred_element_type=jnp.float32)
                  + b2_ref[...]
                  + jnp.dot(var1_ref[...], wr2_ref[...], preferred_element_type=jnp.float32))


def _k_pre2(sa, sb, var1, rdeg, wl2t, wr2t, b2):
    blk = 512
    full = lambda shape: pl.BlockSpec(shape, lambda j: tuple(0 for _ in shape))
    rowblk = pl.BlockSpec((blk, D), lambda j: (j, 0))
    return pl.pallas_call(
        _k_pre2_body,
        grid=(NPAD // blk,),
        out_shape=jax.ShapeDtypeStruct((NPAD, D), jnp.float32),
        in_specs=[rowblk, rowblk, rowblk,
                  pl.BlockSpec((blk, 1), lambda j: (j, 0)),
                  full((D, D)), full((D, D)), full((1, D))],
        out_specs=rowblk,
    )(sa, sb, var1, rdeg, wl2t, wr2t, b2)


SLOTS = 2048
WL_CAP = 440


@functools.lru_cache(maxsize=1)
def _k_scan_fn():
    return functools.partial(
        pl.kernel,
        out_type=jax.ShapeDtypeStruct((2 * 4 * SLOTS, D), jnp.float32),
        mesh=_mesh(),
        compiler_params=pltpu.CompilerParams(needs_layout_passes=False),
        scratch_types=[
            pltpu.VMEM_SHARED((4 * SLOTS, D), jnp.float32),
            pltpu.VMEM((NPAD // 1024, 8, 128), jnp.int32),
            pltpu.VMEM((NPAD // 1024, 8, 128), jnp.int32),
            pltpu.VMEM((10, 8, 128), jnp.int32),
            pltpu.VMEM((10, 8, 128), jnp.int32),
            pltpu.VMEM((520,), jnp.int32),
            pltpu.VMEM((520,), jnp.int32),
            pltpu.VMEM((16, D), jnp.float32),
            pltpu.SemaphoreType.DMA,
        ],
    )(_k_scan_body)


def _k_scan_body(src16, dst16, flagh, sloth, delta, out,
                 acc, flagv, slotv, sbuf, dbuf, wlg, wls, rows, sem):
    core = lax.axis_index("c")
    sid = lax.axis_index("s")

    _zero_vmem_2d(rows, 16)
    for q in range(4 * SLOTS // 16 // 16):
        pltpu.sync_copy(rows, acc.at[pl.ds(sid * (4 * SLOTS // 16) + q * 16, 16)])
    plsc.subcore_barrier()

    pltpu.sync_copy(flagh, flagv)
    pltpu.sync_copy(sloth, slotv)
    pltpu.sync_copy(src16.at[pl.ds(sid * 10, 10)], sbuf)
    pltpu.sync_copy(dst16.at[pl.ds(sid * 10, 10)], dbuf)

    ibase = core * 4

    def batch(t, _):
        gv = wlg[pl.ds(t * 16, 16)]
        sv = wls[pl.ds(t * 16, 16)]
        pltpu.async_copy(delta.at[gv], rows, sem).wait()
        pltpu.sync_copy(rows, acc.at[sv], add=True)
        return 0

    def drain(c):
        nb = c // 16
        lax.fori_loop(0, nb, batch, 0)
        tail_g = wlg[pl.ds(nb * 16, 16)]
        tail_s = wls[pl.ds(nb * 16, 16)]
        wlg[pl.ds(0, 16)] = tail_g
        wls[pl.ds(0, 16)] = tail_s
        return c - nb * 16

    def scan_row(k, cnt):
        srcv = sbuf[k >> 6, (k >> 3) & 7, pl.ds((k & 7) * 16, 16)]
        fv = plsc.load_gather(flagv, [srcv >> 10, (srcv >> 7) & 7, srcv & 127])
        anyf = fv[0]
        for l in range(1, 16):
            anyf = anyf | fv[l]

        def rare(c):
            dstv = dbuf[k >> 6, (k >> 3) & 7, pl.ds((k & 7) * 16, 16)]
            sl = plsc.load_gather(slotv, [dstv >> 10, (dstv >> 7) & 7, dstv & 127])
            for il in range(4):
                ig = ibase + il
                qual = ((fv >> ig) & 1) != 0
                q32 = jnp.where(qual, 1, 0)
                pc = q32[0]
                for l in range(1, 16):
                    pc = pc + q32[l]
                plsc.store_compressed(wlg.at[pl.ds(c, 16)], srcv + ig * NPAD, mask=qual)
                plsc.store_compressed(wls.at[pl.ds(c, 16)], sl + il * SLOTS, mask=qual)
                c = c + pc
            return c

        real = (sid * 640 + k) * 16 < E
        cnt = lax.cond(jnp.logical_and(real, anyf != 0), rare, lambda c: c, cnt)
        cnt = lax.cond(cnt >= WL_CAP, drain, lambda c: c, cnt)
        return cnt

    cnt = lax.fori_loop(0, 640, scan_row, jnp.int32(0))
    wlg[pl.ds(cnt, 16)] = jnp.zeros((16,), jnp.int32)
    wls[pl.ds(cnt, 16)] = jnp.full((16,), 4 * SLOTS - 1, jnp.int32)
    nb = (cnt + 15) // 16
    lax.fori_loop(0, nb, batch, 0)

    plsc.subcore_barrier()
    seg = 4 * SLOTS // 16
    for q in range(seg // 128):
        off = sid * seg + q * 128
        pltpu.sync_copy(acc.at[pl.ds(off, 128)],
                        out.at[pl.ds(core * 4 * SLOTS + off, 128)])


@functools.lru_cache(maxsize=1)
def _k_cgather_fn():
    return functools.partial(
        pl.kernel,
        out_type=(
            jax.ShapeDtypeStruct((NB * SLOTS, D), jnp.float32),
            jax.ShapeDtypeStruct((SLOTS, D), jnp.float32),
            jax.ShapeDtypeStruct((SLOTS,), jnp.float32),
        ),
        mesh=_mesh(),
        compiler_params=pltpu.CompilerParams(needs_layout_passes=False),
        scratch_types=[
            pltpu.VMEM((NPAD // 1024, 8, 128), jnp.int32),
            pltpu.VMEM((NPAD // 1024, 8, 128), jnp.float32),
            pltpu.VMEM((SLOTS // 16, 16), jnp.int32),
            pltpu.VMEM((16, D), jnp.float32),
            pltpu.VMEM((64,), jnp.float32),
            pltpu.SemaphoreType.DMA,
        ],
    )(_k_cgather_body)


def _k_cgather_body(dacc, pre2, sloth, rdegh, cand2d, og, op, or_,
                    slotv, rdegv, candv, rows, rstage, sem):
    w = _wid()
    pltpu.sync_copy(sloth, slotv)
    pltpu.sync_copy(rdegh, rdegv)
    pltpu.sync_copy(cand2d, candv)

    def chunk(ch, _):
        k = w * 4 + ch
        cv = candv[k, :]
        pltpu.async_copy(pre2.at[candv.at[k]], rows, sem).wait()
        pltpu.sync_copy(rows, op.at[pl.ds(k * 16, 16)])
        rv = plsc.load_gather(rdegv, [cv >> 10, (cv >> 7) & 7, cv & 127])
        rstage[pl.ds(ch * 16, 16)] = rv
        sl = plsc.load_gather(slotv, [cv >> 10, (cv >> 7) & 7, cv & 127])
        for i in range(NB):
            pltpu.async_copy(dacc.at[sl + i * SLOTS], rows, sem).wait()
            pltpu.sync_copy(rows, og.at[pl.ds(i * SLOTS + k * 16, 16)])
        return 0

    lax.fori_loop(0, 4, chunk, 0)
    pltpu.sync_copy(rstage, or_.at[pl.ds(w * 64, 64)])


def _k_final_body(bsm_ref, g_ref, p_ref, r_ref, c_ref, dv1_ref, wl2_ref,
                  wr2_ref, o_ref):
    dwr = jnp.dot(dv1_ref[...], wr2_ref[...], preferred_element_type=jnp.float32)
    base = p_ref[...]
    rd = r_ref[...]
    cv = c_ref[...]
    acc = jnp.zeros_like(base)
    for i in range(NB):
        di = jnp.dot(g_ref[i] * rd, wl2_ref[...], preferred_element_type=jnp.float32)
        pre = base + di + jnp.where(cv == bsm_ref[0, i], dwr[i:i + 1, :], 0.0)
        acc = acc + jnp.maximum(pre, 0.0)
    o_ref[...] = acc * (1.0 / NB)


def _k_final(bsm, g, p, r, c, dv1, wl2t, wr2t):
    blk = 256
    full = lambda shape: pl.BlockSpec(shape, lambda j: tuple(0 for _ in shape))
    return pl.pallas_call(
        _k_final_body,
        grid=(SLOTS // blk,),
        out_shape=jax.ShapeDtypeStruct((SLOTS, D), jnp.float32),
        in_specs=[
            pl.BlockSpec(memory_space=pltpu.SMEM),
            pl.BlockSpec((NB, blk, D), lambda j: (0, j, 0)),
            pl.BlockSpec((blk, D), lambda j: (j, 0)),
            pl.BlockSpec((blk, 1), lambda j: (j, 0)),
            pl.BlockSpec((blk, 1), lambda j: (j, 0)),
            full((NB, D)), full((D, D)), full((D, D)),
        ],
        out_specs=pl.BlockSpec((blk, D), lambda j: (j, 0)),
    )(bsm, g, p, r, c, dv1, wl2t, wr2t)


def kernel(variable_embeddings, candidate_indices, constraint_x, variable_x,
           edge_index, edge_attr, params):
    p = params
    src = edge_index[0].astype(jnp.int32)
    dst = edge_index[1].astype(jnp.int32)
    cand = candidate_indices.astype(jnp.int32)

    padrows = (EROWS_PAD - EROWS) * CH
    deadpad = NV + (jnp.arange(padrows, dtype=jnp.int32) % (NPAD - NV))
    src2d = jnp.concatenate([src, deadpad]).reshape(EROWS_PAD, CH)
    dst2d = jnp.concatenate([dst, deadpad]).reshape(EROWS_PAD, CH)
    srcsh2d = src2d + NPAD
    pad16 = K5_ROWS * 16 - E
    src16 = jnp.concatenate([src, jnp.zeros((pad16,), jnp.int32)]).reshape(K5_ROWS, 16)
    dst16 = jnp.concatenate([dst, jnp.full((pad16,), -1, jnp.int32)]).reshape(K5_ROWS, 16)

    def padx(x, k):
        n, f = x.shape
        return jnp.pad(x, ((0, NPAD - n), (0, k - f)))

    cx = padx(constraint_x, 8)
    vx = padx(variable_x, 24)
    csh = jnp.pad(p['cons_shift'], (0, 3)).reshape(1, 8)
    csc = jnp.pad(p['cons_scale'], (0, 3)).reshape(1, 8)
    vsh = jnp.pad(p['var_shift'], (0, 5)).reshape(1, 24)
    vsc = jnp.pad(p['var_scale'], (0, 5)).reshape(1, 24)
    cw1t = jnp.pad(p['cons_W1'].T, ((0, 3), (0, 0)))
    vw1t = jnp.pad(p['var_W1'].T, ((0, 5), (0, 0)))
    cb1 = p['cons_b1'].reshape(1, D)
    cb2 = p['cons_b2'].reshape(1, D)
    vb1 = p['var_b1'].reshape(1, D)
    vb2 = p['var_b2'].reshape(1, D)
    cw2t = p['cons_W2'].T
    vw2t = p['var_W2'].T
    L1, L2 = p['convs'][0], p['convs'][1]
    w = p['break_W'][:, 0].reshape(1, D)

    cnts = _k_counts_fn()(dst2d, srcsh2d)
    bsm, rdeg80, rcnt80 = _k_top8(cnts.reshape(2, 2, NROW, 128))
    rdeg = rdeg80.reshape(NPAD, 1)
    rcnt = rcnt80.reshape(NPAD, 1)
    b8 = bsm[0]
    bvec = jnp.concatenate([b8, jnp.full((8,), 2**30, jnp.int32)])

    cons0 = _k_mlp(cx, csh, csc, cw1t, cb1, cw2t, cb2)
    var0 = _k_mlp(vx, vsh, vsc, vw1t, vb1, vw2t, vb2)

    mparts = _k_mtable_fn()(src16, dst16, bvec)
    ma = mparts[:NPAD * 8].reshape(NPAD, 8)
    mb = mparts[NPAD * 8:].reshape(NPAD, 8)

    src2d_d, _ = lax.optimization_barrier((src2d, mparts))
    scv0 = _k_segsum_fn()(cons0, src2d_d, dst2d)
    dst2d_d, _ = lax.optimization_barrier((dst2d, scv0))
    svc0 = _k_segsum_fn()(var0, dst2d_d, src2d)

    candp = jnp.concatenate([cand, jnp.zeros((48,), jnp.int32)])
    var1, cons1, delta, dv1, flagc, slotmap = _k_layer1(
        bsm, candp.reshape(1, 2048),
        scv0[:NPAD], scv0[NPAD:], svc0[:NPAD], svc0[NPAD:], var0, cons0, rdeg, rcnt,
        ma, mb, w, L1['cv_Wl'].T, L1['cv_Wr'].T, L1['cv_b'].reshape(1, D),
        L1['vc_Wl'].T, L1['vc_Wr'].T, L1['vc_b'].reshape(1, D))

    scv1 = _k_segsum_fn()(cons1, src2d, dst2d)
    pre2 = _k_pre2(scv1[:NPAD], scv1[NPAD:], var1, rdeg,
                   L2['cv_Wl'].T, L2['cv_Wr'].T, L2['cv_b'].reshape(1, D))

    sloth, _ = lax.optimization_barrier((slotmap.reshape(NPAD // 1024, 8, 128), scv1))
    dacc = _k_scan_fn()(src16.reshape(160, 8, 128), dst16.reshape(160, 8, 128),
                        flagc.reshape(NPAD // 1024, 8, 128), sloth,
                        delta.reshape(NPAD * NB, D))

    g, gp, gr = _k_cgather_fn()(dacc, pre2, slotmap.reshape(NPAD // 1024, 8, 128),
                                rdeg80.reshape(NPAD // 1024, 8, 128), candp.reshape(SLOTS // 16, 16))
    res = _k_final(bsm, g.reshape(NB, SLOTS, D), gp, gr.reshape(SLOTS, 1),
                   candp.reshape(SLOTS, 1), dv1, L2['cv_Wl'].T, L2['cv_Wr'].T)
    return res[:NCAND]


# speedup vs baseline: 2.1566x; 1.0092x over previous
import functools

import jax
import jax.numpy as jnp
from jax import lax
from jax.experimental import pallas as pl
from jax.experimental.pallas import tpu as pltpu
from jax.experimental.pallas import tpu_sc as plsc

NV = 10000
NCN = 10000
E = 160000
D = 128
NB = 8
NCAND = 2000

NPAD = 10240
NROW = NPAD // 128
NWRK = 32
CH = 128
EROWS = E // CH
EROWS_PAD = 1280
NCHUNK = EROWS_PAD // NWRK
K5_ROWS = 10240
K5_PW = K5_ROWS // NWRK

@functools.lru_cache(maxsize=1)
def _mesh():
    return plsc.VectorSubcoreMesh(core_axis_name="c", subcore_axis_name="s")


def _wid():
    return lax.axis_index("c") * 16 + lax.axis_index("s")


def _zero_vmem_1d(ref, n):
    z = jnp.zeros((16,), jnp.float32)

    def body(i, _):
        ref[pl.ds(i * 16, 16)] = z
        return 0

    lax.fori_loop(0, n // 16, body, 0)


def _zero_vmem_2d(ref, rows):
    z = jnp.zeros((16,), jnp.float32)

    def body(i, _):
        for c in range(8):
            ref[i, pl.ds(c * 16, 16)] = z
        return 0

    lax.fori_loop(0, rows, body, 0)


@functools.lru_cache(maxsize=1)
def _k_counts_fn():
    return functools.partial(
        pl.kernel,
        out_type=jax.ShapeDtypeStruct((2 * 2 * NPAD,), jnp.float32),
        mesh=_mesh(),
        scratch_types=[
            pltpu.VMEM_SHARED((2 * NPAD,), jnp.float32),
            pltpu.VMEM((NCHUNK, CH), jnp.int32),
            pltpu.VMEM((NCHUNK, CH), jnp.int32),
            pltpu.VMEM((CH,), jnp.float32),
            pltpu.VMEM((2 * NPAD // 16,), jnp.float32),
            pltpu.SemaphoreType.DMA,
        ],
    )(_k_counts_body)


def _k_counts_body(dst2d, srcsh2d, out, tbl, dbuf, sbuf, ones, zbuf, sem):
    core = lax.axis_index("c")
    sid = lax.axis_index("s")
    w = _wid()
    seg = 2 * NPAD // 16

    _zero_vmem_1d(zbuf, seg)
    for v in range(CH // 16):
        ones[pl.ds(v * 16, 16)] = jnp.full((16,), 1.0, jnp.float32)
    pltpu.sync_copy(zbuf, tbl.at[pl.ds(sid * seg, seg)])
    plsc.subcore_barrier()

    pltpu.sync_copy(dst2d.at[pl.ds(w * NCHUNK, NCHUNK)], dbuf)
    pltpu.sync_copy(srcsh2d.at[pl.ds(w * NCHUNK, NCHUNK)], sbuf)

    def body(k2, _):
        d0 = pltpu.async_copy(ones.at[pl.ds(0, CH)], tbl.at[dbuf.at[2 * k2]], sem, add=True)
        d1 = pltpu.async_copy(ones.at[pl.ds(0, CH)], tbl.at[sbuf.at[2 * k2]], sem, add=True)
        d2 = pltpu.async_copy(ones.at[pl.ds(0, CH)], tbl.at[dbuf.at[2 * k2 + 1]], sem, add=True)
        d3 = pltpu.async_copy(ones.at[pl.ds(0, CH)], tbl.at[sbuf.at[2 * k2 + 1]], sem, add=True)
        d0.wait()
        d1.wait()
        d2.wait()
        d3.wait()
        return 0

    lax.fori_loop(0, NCHUNK // 2, body, 0)
    plsc.subcore_barrier()
    pltpu.sync_copy(tbl.at[pl.ds(sid * seg, seg)],
                    out.at[pl.ds(core * 2 * NPAD + sid * seg, seg)])


def _k_top8_body(cnts_ref, b_ref, rdeg_ref, rcnt_ref):
    dsum = cnts_ref[0, 0] + cnts_ref[1, 0]
    csum = cnts_ref[0, 1] + cnts_ref[1, 1]
    r = lax.broadcasted_iota(jnp.int32, (NROW, 128), 0)
    c = lax.broadcasted_iota(jnp.int32, (NROW, 128), 1)
    flat = r * 128 + c
    valid = flat < NV
    d = jnp.where(valid, dsum, -1.0)
    for i in range(NB):
        m = jnp.max(d)
        idx = jnp.min(jnp.where(d == m, flat, jnp.int32(2**30)))
        b_ref[0, i] = idx
        d = jnp.where(flat == idx, -2.0, d)
    rdeg_ref[...] = 1.0 / jnp.maximum(dsum, 1.0)
    rcnt_ref[...] = 1.0 / jnp.maximum(csum, 1.0)


def _k_top8(cnts):
    return pl.pallas_call(
        _k_top8_body,
        out_shape=[
            jax.ShapeDtypeStruct((1, NB), jnp.int32),
            jax.ShapeDtypeStruct((NROW, 128), jnp.float32),
            jax.ShapeDtypeStruct((NROW, 128), jnp.float32),
        ],
        out_specs=[
            pl.BlockSpec(memory_space=pltpu.SMEM),
            pl.BlockSpec((NROW, 128), lambda: (0, 0)),
            pl.BlockSpec((NROW, 128), lambda: (0, 0)),
        ],
        in_specs=[pl.BlockSpec((2, 2, NROW, 128), lambda: (0, 0, 0, 0))],
    )(cnts)


def _k_mlp_body(x_ref, sh_ref, sc_ref, w1_ref, b1_ref, w2_ref, b2_ref, o_ref):
    h = (x_ref[...] + sh_ref[...]) * sc_ref[...]
    h = jnp.maximum(jnp.dot(h, w1_ref[...], preferred_element_type=jnp.float32) + b1_ref[...], 0.0)
    o_ref[...] = jnp.maximum(jnp.dot(h, w2_ref[...], preferred_element_type=jnp.float32) + b2_ref[...], 0.0)


def _k_mlp(x, sh, sc, w1t, b1, w2t, b2):
    k = x.shape[1]
    blk = 512
    return pl.pallas_call(
        _k_mlp_body,
        grid=(NPAD // blk,),
        out_shape=jax.ShapeDtypeStruct((NPAD, D), jnp.float32),
        in_specs=[
            pl.BlockSpec((blk, k), lambda j: (j, 0)),
            pl.BlockSpec((1, k), lambda j: (0, 0)),
            pl.BlockSpec((1, k), lambda j: (0, 0)),
            pl.BlockSpec((k, D), lambda j: (0, 0)),
            pl.BlockSpec((1, D), lambda j: (0, 0)),
            pl.BlockSpec((D, D), lambda j: (0, 0)),
            pl.BlockSpec((1, D), lambda j: (0, 0)),
        ],
        out_specs=pl.BlockSpec((blk, D), lambda j: (j, 0)),
    )(x, sh, sc, w1t, b1, w2t, b2)


@functools.lru_cache(maxsize=1)
def _k_segsum_fn():
    return functools.partial(
        pl.kernel,
        out_type=jax.ShapeDtypeStruct((2 * NPAD, D), jnp.float32),
        mesh=_mesh(),
        scratch_types=[
            pltpu.VMEM_SHARED((NPAD, D), jnp.float32),
            pltpu.VMEM((NCHUNK, CH), jnp.int32),
            pltpu.VMEM((NCHUNK, CH), jnp.int32),
            pltpu.VMEM((CH, D), jnp.float32),
            pltpu.VMEM((CH, D), jnp.float32),
            pltpu.SemaphoreType.DMA,
            pltpu.SemaphoreType.DMA,
            pltpu.SemaphoreType.DMA,
            pltpu.SemaphoreType.DMA,
        ],
    )(_k_segsum_body)


def _k_segsum_body(table, gidx, sidx, out, acc, gbuf, sbuf, rows_a, rows_b,
                   gs_a, gs_b, ss_a, ss_b):
    core = lax.axis_index("c")
    sid = lax.axis_index("s")
    w = _wid()
    bufs = (rows_a, rows_b)
    gsems = (gs_a, gs_b)
    ssems = (ss_a, ss_b)

    _zero_vmem_2d(rows_a, CH)
    for q in range(NPAD // 16 // CH):
        pltpu.sync_copy(rows_a, acc.at[pl.ds(sid * (NPAD // 16) + q * CH, CH)])
    plsc.subcore_barrier()

    pltpu.sync_copy(gidx.at[pl.ds(w * NCHUNK, NCHUNK)], gbuf)
    pltpu.sync_copy(sidx.at[pl.ds(w * NCHUNK, NCHUNK)], sbuf)

    def body(k2, _):
        d0 = pltpu.async_copy(table.at[gbuf.at[2 * k2]], bufs[0], gsems[0])
        d1 = pltpu.async_copy(table.at[gbuf.at[2 * k2 + 1]], bufs[1], gsems[1])
        d0.wait()
        pltpu.sync_copy(bufs[0], acc.at[sbuf.at[2 * k2]], add=True)
        d1.wait()
        pltpu.sync_copy(bufs[1], acc.at[sbuf.at[2 * k2 + 1]], add=True)
        return 0

    lax.fori_loop(0, NCHUNK // 2, body, 0)
    plsc.subcore_barrier()
    for q in range(NPAD // 128 // 16):
        off = sid * (NPAD // 16) + q * 128
        pltpu.sync_copy(acc.at[pl.ds(off, 128)],
                        out.at[pl.ds(core * NPAD + off, 128)])


@functools.lru_cache(maxsize=1)
def _k_mtable_fn():
    return functools.partial(
        pl.kernel,
        out_type=jax.ShapeDtypeStruct((2 * NPAD * 8,), jnp.float32),
        mesh=_mesh(),
        scratch_types=[
            pltpu.VMEM_SHARED((NPAD * 8,), jnp.float32),
            pltpu.VMEM((K5_PW, 16), jnp.int32),
            pltpu.VMEM((K5_PW, 16), jnp.int32),
            pltpu.VMEM((16,), jnp.int32),
            pltpu.VMEM((8, 16), jnp.int32),
            pltpu.VMEM((8, 16), jnp.float32),
            pltpu.VMEM((NPAD * 8 // 16,), jnp.float32),
        ],
    )(_k_mtable_body)


def _k_mtable_body(src16, dst16, bvec, out, msh, sbuf, dbuf, bbuf, istg, vstg, zbuf):
    core = lax.axis_index("c")
    sid = lax.axis_index("s")
    w = _wid()
    seg = NPAD * 8 // 16

    _zero_vmem_1d(zbuf, seg)
    pltpu.sync_copy(zbuf, msh.at[pl.ds(sid * seg, seg)])
    pltpu.sync_copy(bvec, bbuf)
    plsc.subcore_barrier()

    pltpu.sync_copy(src16.at[pl.ds(w * K5_PW, K5_PW)], sbuf)
    pltpu.sync_copy(dst16.at[pl.ds(w * K5_PW, K5_PW)], dbuf)
    bb = bbuf[pl.ds(0, 16)]
    bs = [bb[i] for i in range(NB)]

    def body(k, _):
        dstv = dbuf[k, :]
        srcv = sbuf[k, :]
        hit = dstv == bs[0]
        for i in range(1, NB):
            hit = hit | (dstv == bs[i])
        h32 = jnp.where(hit, 1, 0)
        s = h32[0]
        for l in range(1, 16):
            s = s | h32[l]

        @pl.when(s > 0)
        def _rare():
            base8 = srcv * 8
            for i in range(NB):
                istg[i, :] = base8 + i
                vstg[i, :] = jnp.where(dstv == bs[i], 1.0, 0.0)
            for i in range(NB):
                pltpu.sync_copy(vstg.at[i], msh.at[istg.at[i]], add=True)

        return 0

    lax.fori_loop(0, K5_PW, body, 0)
    plsc.subcore_barrier()
    pltpu.sync_copy(msh.at[pl.ds(sid * seg, seg)],
                    out.at[pl.ds(core * NPAD * 8 + sid * seg, seg)])


def _k_layer1_body(bsm_ref, cand_ref, scv0a_ref, scv0b_ref, svc0a_ref, svc0b_ref,
                   var0_ref, cons0_ref, rdeg_ref, rcnt_ref, ma_ref, mb_ref,
                   w_ref, wl1cv_ref, wr1cv_ref, b1cv_ref, wl1vc_ref,
                   wr1vc_ref, b1vc_ref,
                   var1_ref, cons1_ref, delta_ref, dv1_ref, flag_ref, slot_ref):
    j = pl.program_id(0)
    blk = var0_ref.shape[0]

    a_cv0 = (scv0a_ref[...] + scv0b_ref[...]) * rdeg_ref[...]
    pre_v1 = (jnp.dot(a_cv0, wl1cv_ref[...], preferred_element_type=jnp.float32)
              + b1cv_ref[...]
              + jnp.dot(var0_ref[...], wr1cv_ref[...], preferred_element_type=jnp.float32))
    var1 = jnp.maximum(pre_v1, 0.0)
    var1_ref[...] = var1

    a_vc0 = (svc0a_ref[...] + svc0b_ref[...]) * rcnt_ref[...]
    pre_c1 = (jnp.dot(a_vc0, wl1vc_ref[...], preferred_element_type=jnp.float32)
              + b1vc_ref[...]
              + jnp.dot(cons0_ref[...], wr1vc_ref[...], preferred_element_type=jnp.float32))
    cons1 = jnp.maximum(pre_c1, 0.0)
    cons1_ref[...] = cons1

    u_c = jnp.dot(w_ref[...], wl1vc_ref[...], preferred_element_type=jnp.float32)
    u_v = jnp.dot(w_ref[...], wr1cv_ref[...], preferred_element_type=jnp.float32)

    m = ma_ref[...] + mb_ref[...]
    alpha = m * rcnt_ref[...]
    bits = jnp.zeros((blk, 1), jnp.int32)
    for i in range(NB):
        delta_ref[i] = jnp.maximum(pre_c1 + alpha[:, i:i + 1] * u_c, 0.0) - cons1
        bits = bits + jnp.where(m[:, i:i + 1] > 0.0, jnp.int32(1 << i), 0)
    flag_ref[...] = bits

    rowid = j * blk + lax.broadcasted_iota(jnp.int32, (blk, 1), 0)
    pos = lax.broadcasted_iota(jnp.int32, (1, 2048), 1)
    eq = rowid == cand_ref[...]
    slot_ref[...] = jnp.min(jnp.where(eq, pos, jnp.int32(2047)), axis=1, keepdims=True)

    dblk = jnp.maximum(pre_v1 + u_v, 0.0) - var1

    @pl.when(j == 0)
    def _init():
        dv1_ref[...] = jnp.zeros((NB, D), jnp.float32)

    for i in range(NB):
        sel = rowid == bsm_ref[0, i]
        contrib = jnp.sum(jnp.where(sel, dblk, 0.0), axis=0, keepdims=True)
        dv1_ref[pl.ds(i, 1), :] = dv1_ref[pl.ds(i, 1), :] + contrib


def _k_layer1(bsm, cand2048, scv0a, scv0b, svc0a, svc0b, var0, cons0, rdeg, rcnt,
              ma, mb, w, wl1cv, wr1cv, b1cv, wl1vc, wr1vc, b1vc):
    blk = 512
    g = NPAD // blk
    full = lambda shape: pl.BlockSpec(shape, lambda j: tuple(0 for _ in shape))
    rowblk = pl.BlockSpec((blk, D), lambda j: (j, 0))
    return pl.pallas_call(
        _k_layer1_body,
        grid=(g,),
        out_shape=[
            jax.ShapeDtypeStruct((NPAD, D), jnp.float32),
            jax.ShapeDtypeStruct((NPAD, D), jnp.float32),
            jax.ShapeDtypeStruct((NB, NPAD, D), jnp.float32),
            jax.ShapeDtypeStruct((NB, D), jnp.float32),
            jax.ShapeDtypeStruct((NPAD, 1), jnp.int32),
            jax.ShapeDtypeStruct((NPAD, 1), jnp.int32),
        ],
        in_specs=[
            pl.BlockSpec(memory_space=pltpu.SMEM),
            full((1, 2048)),
            rowblk, rowblk, rowblk, rowblk, rowblk, rowblk,
            pl.BlockSpec((blk, 1), lambda j: (j, 0)),
            pl.BlockSpec((blk, 1), lambda j: (j, 0)),
            pl.BlockSpec((blk, NB), lambda j: (j, 0)),
            pl.BlockSpec((blk, NB), lambda j: (j, 0)),
            full((1, D)), full((D, D)), full((D, D)), full((1, D)),
            full((D, D)), full((D, D)), full((1, D)),
        ],
        out_specs=[
            rowblk, rowblk,
            pl.BlockSpec((NB, blk, D), lambda j: (0, j, 0)),
            pl.BlockSpec((NB, D), lambda j: (0, 0)),
            pl.BlockSpec((blk, 1), lambda j: (j, 0)),
            pl.BlockSpec((blk, 1), lambda j: (j, 0)),
        ],
    )(bsm, cand2048, scv0a, scv0b, svc0a, svc0b, var0, cons0, rdeg, rcnt,
      ma, mb, w, wl1cv, wr1cv, b1cv, wl1vc, wr1vc, b1vc)


def _k_pre2_body(sa_ref, sb_ref, var1_ref, rdeg_ref, wl2_ref, wr2_ref, b2_ref, o_ref):
    a = (sa_ref[...] + sb_ref[...]) * rdeg_ref[...]
    o_ref[...] = (jnp.dot(a, wl2_ref[...], preferred_element_type=jnp.float32)
                  + b2_ref[...]
                  + jnp.dot(var1_ref[...], wr2_ref[...], preferred_element_type=jnp.float32))


def _k_pre2(sa, sb, var1, rdeg, wl2t, wr2t, b2):
    blk = 512
    full = lambda shape: pl.BlockSpec(shape, lambda j: tuple(0 for _ in shape))
    rowblk = pl.BlockSpec((blk, D), lambda j: (j, 0))
    return pl.pallas_call(
        _k_pre2_body,
        grid=(NPAD // blk,),
        out_shape=jax.ShapeDtypeStruct((NPAD, D), jnp.float32),
        in_specs=[rowblk, rowblk, rowblk,
                  pl.BlockSpec((blk, 1), lambda j: (j, 0)),
                  full((D, D)), full((D, D)), full((1, D))],
        out_specs=rowblk,
    )(sa, sb, var1, rdeg, wl2t, wr2t, b2)


SLOTS = 2048
WL_CAP = 440


@functools.lru_cache(maxsize=1)
def _k_scan_fn():
    return functools.partial(
        pl.kernel,
        out_type=jax.ShapeDtypeStruct((2 * 4 * SLOTS, D), jnp.float32),
        mesh=_mesh(),
        compiler_params=pltpu.CompilerParams(needs_layout_passes=False),
        scratch_types=[
            pltpu.VMEM_SHARED((4 * SLOTS, D), jnp.float32),
            pltpu.VMEM((NPAD // 1024, 8, 128), jnp.int32),
            pltpu.VMEM((NPAD // 1024, 8, 128), jnp.int32),
            pltpu.VMEM((10, 8, 128), jnp.int32),
            pltpu.VMEM((10, 8, 128), jnp.int32),
            pltpu.VMEM((520,), jnp.int32),
            pltpu.VMEM((520,), jnp.int32),
            pltpu.VMEM((16, D), jnp.float32),
            pltpu.SemaphoreType.DMA,
        ],
    )(_k_scan_body)


def _k_scan_body(src16, dst16, flagh, sloth, delta, out,
                 acc, flagv, slotv, sbuf, dbuf, wlg, wls, rows, sem):
    core = lax.axis_index("c")
    sid = lax.axis_index("s")

    _zero_vmem_2d(rows, 16)
    for q in range(4 * SLOTS // 16 // 16):
        pltpu.sync_copy(rows, acc.at[pl.ds(sid * (4 * SLOTS // 16) + q * 16, 16)])
    plsc.subcore_barrier()

    pltpu.sync_copy(flagh, flagv)
    pltpu.sync_copy(sloth, slotv)
    pltpu.sync_copy(src16.at[pl.ds(sid * 10, 10)], sbuf)
    pltpu.sync_copy(dst16.at[pl.ds(sid * 10, 10)], dbuf)

    ibase = core * 4

    def batch(t, _):
        gv = wlg[pl.ds(t * 16, 16)]
        sv = wls[pl.ds(t * 16, 16)]
        pltpu.async_copy(delta.at[gv], rows, sem).wait()
        pltpu.sync_copy(rows, acc.at[sv], add=True)
        return 0

    def drain(c):
        nb = c // 16
        lax.fori_loop(0, nb, batch, 0)
        tail_g = wlg[pl.ds(nb * 16, 16)]
        tail_s = wls[pl.ds(nb * 16, 16)]
        wlg[pl.ds(0, 16)] = tail_g
        wls[pl.ds(0, 16)] = tail_s
        return c - nb * 16

    def scan_row(k, cnt):
        srcv = sbuf[k >> 6, (k >> 3) & 7, pl.ds((k & 7) * 16, 16)]
        fv = plsc.load_gather(flagv, [srcv >> 10, (srcv >> 7) & 7, srcv & 127])
        anyf = fv[0]
        for l in range(1, 16):
            anyf = anyf | fv[l]

        def rare(c):
            dstv = dbuf[k >> 6, (k >> 3) & 7, pl.ds((k & 7) * 16, 16)]
            sl = plsc.load_gather(slotv, [dstv >> 10, (dstv >> 7) & 7, dstv & 127])
            for il in range(4):
                ig = ibase + il
                qual = ((fv >> ig) & 1) != 0
                q32 = jnp.where(qual, 1, 0)
                pc = q32[0]
                for l in range(1, 16):
                    pc = pc + q32[l]
                plsc.store_compressed(wlg.at[pl.ds(c, 16)], srcv + ig * NPAD, mask=qual)
                plsc.store_compressed(wls.at[pl.ds(c, 16)], sl + il * SLOTS, mask=qual)
                c = c + pc
            return c

        real = (sid * 640 + k) * 16 < E
        cnt = lax.cond(jnp.logical_and(real, anyf != 0), rare, lambda c: c, cnt)
        cnt = lax.cond(cnt >= WL_CAP, drain, lambda c: c, cnt)
        return cnt

    cnt = lax.fori_loop(0, 640, scan_row, jnp.int32(0))
    wlg[pl.ds(cnt, 16)] = jnp.zeros((16,), jnp.int32)
    wls[pl.ds(cnt, 16)] = jnp.full((16,), 4 * SLOTS - 1, jnp.int32)
    nb = (cnt + 15) // 16
    lax.fori_loop(0, nb, batch, 0)

    plsc.subcore_barrier()
    seg = 4 * SLOTS // 16
    for q in range(seg // 128):
        off = sid * seg + q * 128
        pltpu.sync_copy(acc.at[pl.ds(off, 128)],
                        out.at[pl.ds(core * 4 * SLOTS + off, 128)])


@functools.lru_cache(maxsize=1)
def _k_cgather_fn():
    return functools.partial(
        pl.kernel,
        out_type=(
            jax.ShapeDtypeStruct((NB * SLOTS, D), jnp.float32),
            jax.ShapeDtypeStruct((SLOTS, D), jnp.float32),
            jax.ShapeDtypeStruct((SLOTS,), jnp.float32),
        ),
        mesh=_mesh(),
        compiler_params=pltpu.CompilerParams(needs_layout_passes=False),
        scratch_types=[
            pltpu.VMEM((NPAD // 1024, 8, 128), jnp.int32),
            pltpu.VMEM((NPAD // 1024, 8, 128), jnp.float32),
            pltpu.VMEM((SLOTS // 16, 16), jnp.int32),
            pltpu.VMEM((16, D), jnp.float32),
            pltpu.VMEM((64,), jnp.float32),
            pltpu.SemaphoreType.DMA,
        ],
    )(_k_cgather_body)


def _k_cgather_body(dacc, pre2, sloth, rdegh, cand2d, og, op, or_,
                    slotv, rdegv, candv, rows, rstage, sem):
    w = _wid()
    pltpu.sync_copy(sloth, slotv)
    pltpu.sync_copy(rdegh, rdegv)
    pltpu.sync_copy(cand2d, candv)

    def chunk(ch, _):
        k = w * 4 + ch
        cv = candv[k, :]
        pltpu.async_copy(pre2.at[candv.at[k]], rows, sem).wait()
        pltpu.sync_copy(rows, op.at[pl.ds(k * 16, 16)])
        rv = plsc.load_gather(rdegv, [cv >> 10, (cv >> 7) & 7, cv & 127])
        rstage[pl.ds(ch * 16, 16)] = rv
        sl = plsc.load_gather(slotv, [cv >> 10, (cv >> 7) & 7, cv & 127])
        for i in range(NB):
            pltpu.async_copy(dacc.at[sl + i * SLOTS], rows, sem).wait()
            pltpu.sync_copy(rows, og.at[pl.ds(i * SLOTS + k * 16, 16)])
        return 0

    lax.fori_loop(0, 4, chunk, 0)
    pltpu.sync_copy(rstage, or_.at[pl.ds(w * 64, 64)])


def _k_final_body(bsm_ref, g_ref, p_ref, r_ref, c_ref, dv1_ref, wl2_ref,
                  wr2_ref, o_ref):
    dwr = jnp.dot(dv1_ref[...], wr2_ref[...], preferred_element_type=jnp.float32)
    base = p_ref[...]
    rd = r_ref[...]
    cv = c_ref[...]
    acc = jnp.zeros_like(base)
    for i in range(NB):
        di = jnp.dot(g_ref[i] * rd, wl2_ref[...], preferred_element_type=jnp.float32)
        pre = base + di + jnp.where(cv == bsm_ref[0, i], dwr[i:i + 1, :], 0.0)
        acc = acc + jnp.maximum(pre, 0.0)
    o_ref[...] = acc * (1.0 / NB)


def _k_final(bsm, g, p, r, c, dv1, wl2t, wr2t):
    blk = 256
    full = lambda shape: pl.BlockSpec(shape, lambda j: tuple(0 for _ in shape))
    return pl.pallas_call(
        _k_final_body,
        grid=(SLOTS // blk,),
        out_shape=jax.ShapeDtypeStruct((SLOTS, D), jnp.float32),
        in_specs=[
            pl.BlockSpec(memory_space=pltpu.SMEM),
            pl.BlockSpec((NB, blk, D), lambda j: (0, j, 0)),
            pl.BlockSpec((blk, D), lambda j: (j, 0)),
            pl.BlockSpec((blk, 1), lambda j: (j, 0)),
            pl.BlockSpec((blk, 1), lambda j: (j, 0)),
            full((NB, D)), full((D, D)), full((D, D)),
        ],
        out_specs=pl.BlockSpec((blk, D), lambda j: (j, 0)),
    )(bsm, g, p, r, c, dv1, wl2t, wr2t)


def kernel(variable_embeddings, candidate_indices, constraint_x, variable_x,
           edge_index, edge_attr, params):
    p = params
    src = edge_index[0].astype(jnp.int32)
    dst = edge_index[1].astype(jnp.int32)
    cand = candidate_indices.astype(jnp.int32)

    padrows = (EROWS_PAD - EROWS) * CH
    deadpad = NV + (jnp.arange(padrows, dtype=jnp.int32) % (NPAD - NV))
    src2d = jnp.concatenate([src, deadpad]).reshape(EROWS_PAD, CH)
    dst2d = jnp.concatenate([dst, deadpad]).reshape(EROWS_PAD, CH)
    srcsh2d = src2d + NPAD
    pad16 = K5_ROWS * 16 - E
    src16 = jnp.concatenate([src, jnp.zeros((pad16,), jnp.int32)]).reshape(K5_ROWS, 16)
    dst16 = jnp.concatenate([dst, jnp.full((pad16,), -1, jnp.int32)]).reshape(K5_ROWS, 16)

    def padx(x, k):
        n, f = x.shape
        return jnp.pad(x, ((0, NPAD - n), (0, k - f)))

    cx = padx(constraint_x, 8)
    vx = padx(variable_x, 24)
    csh = jnp.pad(p['cons_shift'], (0, 3)).reshape(1, 8)
    csc = jnp.pad(p['cons_scale'], (0, 3)).reshape(1, 8)
    vsh = jnp.pad(p['var_shift'], (0, 5)).reshape(1, 24)
    vsc = jnp.pad(p['var_scale'], (0, 5)).reshape(1, 24)
    cw1t = jnp.pad(p['cons_W1'].T, ((0, 3), (0, 0)))
    vw1t = jnp.pad(p['var_W1'].T, ((0, 5), (0, 0)))
    cb1 = p['cons_b1'].reshape(1, D)
    cb2 = p['cons_b2'].reshape(1, D)
    vb1 = p['var_b1'].reshape(1, D)
    vb2 = p['var_b2'].reshape(1, D)
    cw2t = p['cons_W2'].T
    vw2t = p['var_W2'].T
    L1, L2 = p['convs'][0], p['convs'][1]
    w = p['break_W'][:, 0].reshape(1, D)

    cnts = _k_counts_fn()(dst2d, srcsh2d)
    bsm, rdeg80, rcnt80 = _k_top8(cnts.reshape(2, 2, NROW, 128))
    rdeg = rdeg80.reshape(NPAD, 1)
    rcnt = rcnt80.reshape(NPAD, 1)
    b8 = bsm[0]
    bvec = jnp.concatenate([b8, jnp.full((8,), 2**30, jnp.int32)])

    cons0 = _k_mlp(cx, csh, csc, cw1t, cb1, cw2t, cb2)
    var0 = _k_mlp(vx, vsh, vsc, vw1t, vb1, vw2t, vb2)

    mparts = _k_mtable_fn()(src16, dst16, bvec)
    ma = mparts[:NPAD * 8].reshape(NPAD, 8)
    mb = mparts[NPAD * 8:].reshape(NPAD, 8)

    src2d_d, _ = lax.optimization_barrier((src2d, mparts))
    scv0 = _k_segsum_fn()(cons0, src2d_d, dst2d)
    dst2d_d, _ = lax.optimization_barrier((dst2d, scv0))
    svc0 = _k_segsum_fn()(var0, dst2d_d, src2d)

    candp = jnp.concatenate([cand, jnp.zeros((48,), jnp.int32)])
    var1, cons1, delta, dv1, flagc, slotmap = _k_layer1(
        bsm, candp.reshape(1, 2048),
        scv0[:NPAD], scv0[NPAD:], svc0[:NPAD], svc0[NPAD:], var0, cons0, rdeg, rcnt,
        ma, mb, w, L1['cv_Wl'].T, L1['cv_Wr'].T, L1['cv_b'].reshape(1, D),
        L1['vc_Wl'].T, L1['vc_Wr'].T, L1['vc_b'].reshape(1, D))

    scv1 = _k_segsum_fn()(cons1, src2d, dst2d)
    pre2 = _k_pre2(scv1[:NPAD], scv1[NPAD:], var1, rdeg,
                   L2['cv_Wl'].T, L2['cv_Wr'].T, L2['cv_b'].reshape(1, D))

    sloth, _ = lax.optimization_barrier((slotmap.reshape(NPAD // 1024, 8, 128), scv1))
    dacc = _k_scan_fn()(src16.reshape(160, 8, 128), dst16.reshape(160, 8, 128),
                        flagc.reshape(NPAD // 1024, 8, 128), sloth,
                        delta.reshape(NPAD * NB, D))

    g, gp, gr = _k_cgather_fn()(dacc, pre2, slotmap.reshape(NPAD // 1024, 8, 128),
                                rdeg80.reshape(NPAD // 1024, 8, 128), candp.reshape(SLOTS // 16, 16))
    res = _k_final(bsm, g.reshape(NB, SLOTS, D), gp, gr.reshape(SLOTS, 1),
                   candp.reshape(SLOTS, 1), dv1, L2['cv_Wl'].T, L2['cv_Wr'].T)
    return res[:NCAND]
